# Initial kernel scaffold; baseline (speedup 1.0000x reference)
import jax
import jax.numpy as jnp
from jax.experimental import pallas as pl


def kernel(query, key, value, Wq, bq, Wk, bk, Wv, bv, Wo, bo, coarse_temperature):
    raise NotImplementedError("write your pallas kernel here")


# speedup vs baseline: 25.1317x; 25.1317x over previous
import functools

import jax
import jax.numpy as jnp
from jax.experimental import pallas as pl
from jax.experimental.pallas import tpu as pltpu

H = 12
D = 64
W = 64
K = 8
NEG_INF = float("-inf")


def _k1_body(qin, kin, vin, wqt, wkt, wvt, bq2, bk2, bv2,
             qout, kout, vout, qsum):
    sb = pl.program_id(1)
    x = qin[0]
    qout[0] = jnp.dot(x, wqt[...], preferred_element_type=jnp.float32) + bq2[...]
    kfull = jnp.dot(kin[0], wkt[...], preferred_element_type=jnp.float32) + bk2[...]
    vfull = jnp.dot(vin[0], wvt[...], preferred_element_type=jnp.float32) + bv2[...]
    for h in range(H):
        kout[0, h] = kfull[:, h * D:(h + 1) * D]
        vout[0, h] = vfull[:, h * D:(h + 1) * D]
    cs = jnp.sum(x, axis=0, keepdims=True)

    @pl.when(sb == 0)
    def _():
        qsum[0] = cs

    @pl.when(sb != 0)
    def _():
        qsum[0] = qsum[0] + cs


def _k2_body(S, kin, qsum, wqt, bq2, ct, sout):
    qc = jnp.dot(qsum[0] * (1.0 / S), wqt[...],
                 preferred_element_type=jnp.float32) + bq2[...]
    blkS = kin.shape[2]
    nwb = blkS // W
    wrow = jax.lax.broadcasted_iota(jnp.int32, (nwb, blkS), 0)
    wcol = jax.lax.broadcasted_iota(jnp.int32, (nwb, blkS), 1) // W
    WinM = (wrow == wcol).astype(jnp.float32)
    cols = []
    for h in range(H):
        qch = qc[:, h * D:(h + 1) * D]
        qn2 = jnp.sum(qch * qch, axis=1, keepdims=True)
        qn = qch * (1.0 / jnp.maximum(jnp.sqrt(qn2), 1e-8))
        kh = kin[0, h]
        d = jnp.sum(kh * qn, axis=1, keepdims=True)
        n2 = jnp.sum(kh * kh, axis=1, keepdims=True)
        invk = 1.0 / jnp.maximum(jnp.sqrt(n2), 1e-8)
        c = (d * invk) ** 2
        cols.append(jnp.dot(WinM, c, preferred_element_type=jnp.float32))
    ws = jnp.concatenate(cols, axis=1)
    temp = jnp.maximum(ct[0, 0], 0.01)
    sout[0] = ws * (1.0 / (W * temp))


def _ktk_body(kk, sref, idx_out, ca_out):
    s = sref[...]
    BH, nw = s.shape
    iota = jax.lax.broadcasted_iota(jnp.int32, (BH, nw), 1)
    vals, idxs = [], []
    for _ in range(kk):
        m = jnp.max(s, axis=1, keepdims=True)
        is_m = s == m
        sel = jnp.min(jnp.where(is_m, iota, nw), axis=1, keepdims=True)
        vals.append(m)
        idxs.append(sel)
        s = jnp.where(iota == sel, NEG_INF, s)
    V = jnp.concatenate(vals, axis=1)
    I = jnp.concatenate(idxs, axis=1)
    mm = jnp.max(V, axis=1, keepdims=True)
    e = jnp.exp(V - mm)
    ca = e / jnp.sum(e, axis=1, keepdims=True)
    idx_out[...] = I
    ca_out[...] = ca


def _kg_body(kk, idx_ref, ca_ref, k_any, v_any, kf_ref, vf_ref, sems):
    g = pl.program_id(0)
    copies = []
    for j in range(kk):
        w = idx_ref[g * kk + j]
        src_k = k_any.at[g, pl.ds(w * W, W), :]
        cp = pltpu.make_async_copy(src_k, kf_ref.at[0, pl.ds(j * W, W), :],
                                   sems.at[j])
        cp.start()
        copies.append(cp)
        src_v = v_any.at[g, pl.ds(w * W, W), :]
        cpv = pltpu.make_async_copy(src_v, vf_ref.at[0, pl.ds(j * W, W), :],
                                    sems.at[kk + j])
        cpv.start()
        copies.append(cpv)
    for cp in copies:
        cp.wait()
    for j in range(kk):
        ca = ca_ref[g * kk + j]
        vf_ref[0, j * W:(j + 1) * W, :] = vf_ref[0, j * W:(j + 1) * W, :] * ca


def _k3_body(scale, q_ref, kf_ref, vf_ref, wot, bo2, out_ref):
    blkQ = q_ref.shape[1]
    E = wot.shape[0]
    qb = q_ref[0]
    acc = jnp.broadcast_to(bo2[...], (blkQ, E))
    for h in range(H):
        qh = qb[:, h * D:(h + 1) * D]
        kfh = kf_ref[0, h]
        s = jax.lax.dot_general(qh, kfh, (((1,), (1,)), ((), ())),
                                preferred_element_type=jnp.float32) * scale
        m = jnp.max(s, axis=1, keepdims=True)
        p = jnp.exp(s - m)
        z = jnp.sum(p, axis=1, keepdims=True)
        o = jnp.dot(p, vf_ref[0, h], preferred_element_type=jnp.float32) / z
        acc = acc + jnp.dot(o, wot[h * D:(h + 1) * D, :],
                            preferred_element_type=jnp.float32)
    out_ref[0] = acc


def kernel(query, key, value, Wq, bq, Wk, bk, Wv, bv, Wo, bo,
           coarse_temperature):
    B, S, E = query.shape
    nw = S // W
    kk = min(K, nw)
    L = kk * W
    scale = D ** -0.5

    wqt, wkt, wvt, wot = Wq.T, Wk.T, Wv.T, Wo.T
    bq2, bk2, bv2, bo2 = (x.reshape(1, E) for x in (bq, bk, bv, bo))
    ct = coarse_temperature.reshape(1, 1)

    blkS = min(512, S)
    nS = S // blkS

    q, k, v, qsum = pl.pallas_call(
        _k1_body,
        grid=(B, nS),
        in_specs=[
            pl.BlockSpec((1, blkS, E), lambda b, s: (b, s, 0)),
            pl.BlockSpec((1, blkS, E), lambda b, s: (b, s, 0)),
            pl.BlockSpec((1, blkS, E), lambda b, s: (b, s, 0)),
            pl.BlockSpec((E, E), lambda b, s: (0, 0)),
            pl.BlockSpec((E, E), lambda b, s: (0, 0)),
            pl.BlockSpec((E, E), lambda b, s: (0, 0)),
            pl.BlockSpec((1, E), lambda b, s: (0, 0)),
            pl.BlockSpec((1, E), lambda b, s: (0, 0)),
            pl.BlockSpec((1, E), lambda b, s: (0, 0)),
        ],
        out_specs=[
            pl.BlockSpec((1, blkS, E), lambda b, s: (b, s, 0)),
            pl.BlockSpec((1, H, blkS, D), lambda b, s: (b, 0, s, 0)),
            pl.BlockSpec((1, H, blkS, D), lambda b, s: (b, 0, s, 0)),
            pl.BlockSpec((1, 1, E), lambda b, s: (b, 0, 0)),
        ],
        out_shape=[
            jax.ShapeDtypeStruct((B, S, E), jnp.float32),
            jax.ShapeDtypeStruct((B, H, S, D), jnp.float32),
            jax.ShapeDtypeStruct((B, H, S, D), jnp.float32),
            jax.ShapeDtypeStruct((B, 1, E), jnp.float32),
        ],
    )(query, key, value, wqt, wkt, wvt, bq2, bk2, bv2)

    nwb = blkS // W
    scores = pl.pallas_call(
        functools.partial(_k2_body, S),
        grid=(B, nS),
        in_specs=[
            pl.BlockSpec((1, H, blkS, D), lambda b, s: (b, 0, s, 0)),
            pl.BlockSpec((1, 1, E), lambda b, s: (b, 0, 0)),
            pl.BlockSpec((E, E), lambda b, s: (0, 0)),
            pl.BlockSpec((1, E), lambda b, s: (0, 0)),
            pl.BlockSpec((1, 1), lambda b, s: (0, 0),
                         memory_space=pltpu.SMEM),
        ],
        out_specs=pl.BlockSpec((1, nwb, H), lambda b, s: (b, s, 0)),
        out_shape=jax.ShapeDtypeStruct((B, nw, H), jnp.float32),
    )(k, qsum, wqt, bq2, ct)

    s_bh = scores.transpose(0, 2, 1).reshape(B * H, nw)
    top_idx, ca = pl.pallas_call(
        functools.partial(_ktk_body, kk),
        out_shape=[
            jax.ShapeDtypeStruct((B * H, kk), jnp.int32),
            jax.ShapeDtypeStruct((B * H, kk), jnp.float32),
        ],
    )(s_bh)

    idx_flat = top_idx.reshape(-1)
    ca_flat = ca.reshape(-1)
    kf, vf = pl.pallas_call(
        functools.partial(_kg_body, kk),
        grid_spec=pltpu.PrefetchScalarGridSpec(
            num_scalar_prefetch=2,
            grid=(B * H,),
            in_specs=[
                pl.BlockSpec(memory_space=pl.ANY),
                pl.BlockSpec(memory_space=pl.ANY),
            ],
            out_specs=[
                pl.BlockSpec((1, L, D), lambda g, i, c: (g, 0, 0)),
                pl.BlockSpec((1, L, D), lambda g, i, c: (g, 0, 0)),
            ],
            scratch_shapes=[pltpu.SemaphoreType.DMA((2 * kk,))],
        ),
        out_shape=[
            jax.ShapeDtypeStruct((B * H, L, D), jnp.float32),
            jax.ShapeDtypeStruct((B * H, L, D), jnp.float32),
        ],
    )(idx_flat, ca_flat, k.reshape(B * H, S, D), v.reshape(B * H, S, D))
    kf = kf.reshape(B, H, L, D)
    vf = vf.reshape(B, H, L, D)

    blkQ = min(512, S)
    nQ = S // blkQ
    out = pl.pallas_call(
        functools.partial(_k3_body, scale),
        grid=(B, nQ),
        in_specs=[
            pl.BlockSpec((1, blkQ, E), lambda b, qb: (b, qb, 0)),
            pl.BlockSpec((1, H, L, D), lambda b, qb: (b, 0, 0, 0)),
            pl.BlockSpec((1, H, L, D), lambda b, qb: (b, 0, 0, 0)),
            pl.BlockSpec((E, E), lambda b, qb: (0, 0)),
            pl.BlockSpec((1, E), lambda b, qb: (0, 0)),
        ],
        out_specs=pl.BlockSpec((1, blkQ, E), lambda b, qb: (b, qb, 0)),
        out_shape=jax.ShapeDtypeStruct((B, S, E), jnp.float32),
    )(q, kf, vf, wot, bo2)
    return out
